# baseline (speedup 1.0000x reference)
import functools

import jax
import jax.numpy as jnp
from jax.experimental import pallas as pl
from jax.experimental.pallas import tpu as pltpu

_N = 5000
_M = 32
_TOT = _N + _M
_ROWS = 40
_PAD = _ROWS * 128
_K_FG = 128
_K_BG = 384


def _body(gt_ref, cls_ref, x1_ref, y1_ref, x2_ref, y2_ref, sc_ref,
          idx_ref, cls_out_ref, iou_out_ref):
    f32 = jnp.float32
    ii = (jax.lax.broadcasted_iota(jnp.int32, (_ROWS, 128), 0) * 128
          + jax.lax.broadcasted_iota(jnp.int32, (_ROWS, 128), 1))
    valid = ii < _TOT

    px1 = x1_ref[...]
    py1 = y1_ref[...]
    px2 = x2_ref[...]
    py2 = y2_ref[...]
    score = sc_ref[...]
    parea = (px2 - px1) * (py2 - py1)

    best_v = jnp.full((_ROWS, 128), -1.0, f32)
    best_c = jnp.zeros((_ROWS, 128), jnp.int32)
    for g in range(_M):
        gx1 = gt_ref[g, 0]
        gy1 = gt_ref[g, 1]
        gx2 = gt_ref[g, 2]
        gy2 = gt_ref[g, 3]
        garea = (gx2 - gx1) * (gy2 - gy1)
        iw = jnp.maximum(jnp.minimum(px2, gx2) - jnp.maximum(px1, gx1), 0.0)
        ih = jnp.maximum(jnp.minimum(py2, gy2) - jnp.maximum(py1, gy1), 0.0)
        inter = iw * ih
        iou = inter / (parea + garea - inter + 1e-9)
        better = iou > best_v
        best_v = jnp.where(better, iou, best_v)
        best_c = jnp.where(better, cls_ref[g], best_c)

    label1 = best_v >= 0.7
    labelm1 = jnp.logical_and(jnp.logical_not(label1), best_v >= 0.3)
    gcls = jnp.where(label1, best_c,
                     jnp.where(labelm1, -1, jnp.int32(1)))
    is_fg = jnp.logical_and(gcls >= 0, gcls < 1)
    is_bg = gcls == 1
    prio = best_v + 1e-4 * score
    fgk = jnp.where(jnp.logical_and(valid, is_fg), prio, -1.0)
    bgk = jnp.where(jnp.logical_and(valid, is_bg), -prio, -10.0)

    def extract(A, k, rows, kill):
        lane = (jax.lax.broadcasted_iota(jnp.int32, (rows, 128), 0) * 128
                + jax.lax.broadcasted_iota(jnp.int32, (rows, 128), 1))
        oi = jnp.zeros((rows, 128), jnp.int32)
        oc = jnp.zeros((rows, 128), jnp.int32)
        ov = jnp.zeros((rows, 128), f32)

        def step(s, carry):
            A, oi, oc, ov = carry
            m = jnp.max(A)
            cand = jnp.where(A == m, ii, jnp.int32(99999))
            sel = jnp.min(cand)
            hit = ii == sel
            c = jnp.max(jnp.where(hit, gcls, jnp.int32(-2)))
            v = jnp.max(jnp.where(hit, best_v, -1.0))
            put = lane == s
            oi = jnp.where(put, sel, oi)
            oc = jnp.where(put, c, oc)
            ov = jnp.where(put, v, ov)
            A = jnp.where(hit, kill, A)
            return A, oi, oc, ov

        A, oi, oc, ov = jax.lax.fori_loop(0, k, step, (A, oi, oc, ov))
        return oi, oc, ov

    fi, fc, fv = extract(fgk, _K_FG, 1, -2.0)
    bi, bc, bv = extract(bgk, _K_BG, 3, -11.0)
    idx_ref[...] = jnp.concatenate([fi, bi], axis=0)
    cls_out_ref[...] = jnp.concatenate([fc, bc], axis=0)
    iou_out_ref[...] = jnp.concatenate([fv, bv], axis=0)


@jax.jit
def kernel(proposal_boxes, proposal_scores, gt_boxes, gt_classes):
    boxes = jnp.concatenate([proposal_boxes, gt_boxes], axis=0)
    boxes = jnp.pad(boxes, ((0, _PAD - _TOT), (0, 0)))
    scores = jnp.concatenate(
        [proposal_scores, jnp.ones((_M,), jnp.float32)], axis=0)
    scores = jnp.pad(scores, (0, _PAD - _TOT)).reshape(_ROWS, 128)
    cols = [boxes[:, i].reshape(_ROWS, 128) for i in range(4)]
    gt_classes = gt_classes.astype(jnp.int32)

    smem = pl.BlockSpec(memory_space=pltpu.SMEM)
    out = pl.pallas_call(
        _body,
        in_specs=[smem, smem] + [pl.BlockSpec((_ROWS, 128), lambda: (0, 0))] * 5,
        out_specs=[pl.BlockSpec((4, 128), lambda: (0, 0))] * 3,
        out_shape=(
            jax.ShapeDtypeStruct((4, 128), jnp.int32),
            jax.ShapeDtypeStruct((4, 128), jnp.int32),
            jax.ShapeDtypeStruct((4, 128), jnp.float32),
        ),
    )(gt_boxes, gt_classes, *cols, scores)
    return (out[0].reshape(512), out[1].reshape(512), out[2].reshape(512))


# speedup vs baseline: 6.9453x; 6.9453x over previous
import functools

import jax
import jax.numpy as jnp
from jax import lax
from jax.experimental import pallas as pl
from jax.experimental.pallas import tpu as pltpu
from jax.experimental.pallas import tpu_sc as plsc

_N = 5000
_M = 32
_TOT = _N + _M
_ROWS = 40
_PAD = _ROWS * 128
_K_FG = 128
_K_BG = 384
_K = 512
_NT = 16
_E = _PAD // _NT
_EV = _E // 16
_LCAP = 640
_SENT = 0x7FFFFFFF
_IMIN = -2147483648


def _match_body(gt_ref, cls_ref, x1_ref, y1_ref, x2_ref, y2_ref, sc_ref,
                vf_ref, vb_ref, mv_ref, gc_ref):
    f32 = jnp.float32
    ii = (lax.broadcasted_iota(jnp.int32, (_ROWS, 128), 0) * 128
          + lax.broadcasted_iota(jnp.int32, (_ROWS, 128), 1))
    valid = ii < _TOT

    px1 = x1_ref[...]
    py1 = y1_ref[...]
    px2 = x2_ref[...]
    py2 = y2_ref[...]
    score = sc_ref[...]
    parea = (px2 - px1) * (py2 - py1)

    best_v = jnp.full((_ROWS, 128), -1.0, f32)
    best_c = jnp.zeros((_ROWS, 128), jnp.int32)
    for g in range(_M):
        gx1 = gt_ref[g, 0]
        gy1 = gt_ref[g, 1]
        gx2 = gt_ref[g, 2]
        gy2 = gt_ref[g, 3]
        garea = (gx2 - gx1) * (gy2 - gy1)
        iw = jnp.maximum(jnp.minimum(px2, gx2) - jnp.maximum(px1, gx1), 0.0)
        ih = jnp.maximum(jnp.minimum(py2, gy2) - jnp.maximum(py1, gy1), 0.0)
        inter = iw * ih
        iou = inter / (parea + garea - inter + 1e-9)
        better = iou > best_v
        best_v = jnp.where(better, iou, best_v)
        best_c = jnp.where(better, cls_ref[g], best_c)

    label1 = best_v >= 0.7
    labelm1 = jnp.logical_and(jnp.logical_not(label1), best_v >= 0.3)
    gcls = jnp.where(label1, best_c, jnp.where(labelm1, -1, jnp.int32(1)))
    is_fg = jnp.logical_and(gcls >= 0, gcls < 1)
    is_bg = gcls == 1
    prio = best_v + 1e-4 * score
    fgk = jnp.where(jnp.logical_and(valid, is_fg), prio, -1.0)
    bgk = jnp.where(jnp.logical_and(valid, is_bg), -prio, -10.0)

    def sortable(x):
        u = lax.bitcast_convert_type(x, jnp.int32)
        s = jnp.where(u < 0, ~u, u | jnp.int32(_IMIN))
        return jnp.where(valid, s, 0)

    vf_ref[...] = sortable(fgk)
    vb_ref[...] = sortable(bgk)
    mv_ref[...] = best_v
    gc_ref[...] = gcls


def _u32(x):
    return plsc.bitcast(x, jnp.uint32)


def _select_body(vf_hbm, vb_hbm, mv_hbm, gc_hbm,
                 oi_hbm, oc_hbm, ov_hbm,
                 va, vb2, mva, gca, hist, htab,
                 la_v, la_i, la_mv, la_gc,
                 lv, li, lmv, lgc, mgi, mgf, stf, msem,
                 cbuf, ctab, obi, obc, obv,
                 sh_hist, sh_cnt, sh_lv, sh_li,
                 sh_oi, sh_oc, sh_ov):
    i32 = jnp.int32
    u32 = jnp.uint32
    wid = lax.axis_index("s")
    cid = lax.axis_index("c")
    fgc = cid == 0
    k = jnp.where(fgc, _K_FG, _K_BG)
    obase = jnp.where(fgc, 0, _K_FG)
    base = wid * _E
    iota = lax.broadcasted_iota(i32, (16,), 0)
    ones = jnp.ones((16,), i32)

    pltpu.sync_copy(vf_hbm.at[pl.ds(base, _E)], va)
    pltpu.sync_copy(vb_hbm.at[pl.ds(base, _E)], vb2)
    pltpu.sync_copy(mv_hbm.at[pl.ds(base, _E)], mva)
    pltpu.sync_copy(gc_hbm.at[pl.ds(base, _E)], gca)

    fgcv = (jnp.zeros((16,), jnp.int32) + cid) == 0

    def key_sel(j):
        f = va[pl.ds(j * 16, 16)]
        b = vb2[pl.ds(j * 16, 16)]
        return _u32(jnp.where(fgcv, f, b))

    pfx = jnp.uint32(0)
    k_rem = k
    for l in range(4):
        shift = jnp.uint32(24 - 8 * l)
        himask = jnp.uint32((0xFFFFFFFF << (32 - 8 * l)) & 0xFFFFFFFF)

        def zb(j, _):
            hist[pl.ds(j * 16, 16)] = jnp.zeros((16,), i32)
            return 0
        lax.fori_loop(0, 16, zb, 0)

        def hb(j, _, pfx=pfx, shift=shift, himask=himask, lvl=l):
            v = key_sel(j)
            d = ((v >> shift) & jnp.uint32(0xFF)).astype(i32)
            if lvl == 0:
                plsc.addupdate_scatter(hist, [d], ones)
            else:
                act = (v & himask) == pfx
                plsc.addupdate_scatter(hist, [d], ones, mask=act)
            return 0
        lax.fori_loop(0, _EV, hb, 0)

        pltpu.sync_copy(hist, sh_hist.at[l, wid])
        plsc.subcore_barrier()
        pltpu.sync_copy(sh_hist.at[l], htab)

        def sb(jj, carry, k_rem=k_rem):
            chi, bst, sst = carry
            j = 15 - jj
            h = htab[0, pl.ds(j * 16, 16)]
            for t in range(1, _NT):
                h = h + htab[t, pl.ds(j * 16, 16)]
            incl = plsc.cumsum(h)
            total = jnp.sum(h)
            s_vec = chi + total - incl
            cond = jnp.logical_and(s_vec < k_rem, s_vec + h >= k_rem)
            bst = jnp.maximum(bst, jnp.max(jnp.where(cond, iota + j * 16, -1)))
            sst = jnp.maximum(sst, jnp.max(jnp.where(cond, s_vec, -1)))
            return chi + total, bst, sst
        _, bstar, sstar = lax.fori_loop(0, 16, sb, (jnp.int32(0), jnp.int32(-1), jnp.int32(-1)))
        pfx = pfx | (bstar.astype(u32) << shift)
        k_rem = k_rem - sstar

    vstar = pfx
    big_r = k_rem
    a_cnt = k - big_r

    def pf(j, _):
        la_v[pl.ds(j * 16, 16)] = jnp.zeros((16,), i32)
        la_i[pl.ds(j * 16, 16)] = jnp.full((16,), _SENT, i32)
        return 0
    lax.fori_loop(0, 22, pf, 0)

    def cb(j, carry):
        n_ab, n_tie = carry
        v = key_sel(j)
        m = v > vstar
        mi = m.astype(i32)
        pos = n_ab + plsc.cumsum(mi) - mi
        gi = base + j * 16 + iota
        plsc.store_scatter(la_v, [pos], plsc.bitcast(v, i32), mask=m)
        plsc.store_scatter(la_i, [pos], gi, mask=m)
        plsc.store_scatter(la_mv, [pos], mva[pl.ds(j * 16, 16)], mask=m)
        plsc.store_scatter(la_gc, [pos], gca[pl.ds(j * 16, 16)], mask=m)
        n_ab = n_ab + jnp.max(plsc.all_reduce_population_count(m))
        n_tie = n_tie + jnp.max(plsc.all_reduce_population_count(v == vstar))
        return n_ab, n_tie
    n_above, n_tie = lax.fori_loop(0, _EV, cb, (jnp.int32(0), jnp.int32(0)))

    cbuf[...] = jnp.where(iota == 0, n_above, jnp.where(iota == 1, n_tie, 0))
    pltpu.sync_copy(cbuf, sh_cnt.at[wid])
    plsc.subcore_barrier()
    pltpu.sync_copy(sh_cnt, ctab)
    zer = jnp.zeros((16,), i32)
    av = plsc.load_gather(ctab, [iota, zer])
    tv = plsc.load_gather(ctab, [iota, zer + 1])
    chunks = lax.shift_right_arithmetic(av + 15, 4)
    entries = chunks * 16
    ecum = plsc.cumsum(entries)
    my_wbase = jnp.max(jnp.where(iota == wid, ecum - entries, 0))
    my_chunks = jnp.max(jnp.where(iota == wid, chunks, 0))
    tcum = plsc.cumsum(tv)
    my_tiebase = jnp.max(jnp.where(iota == wid, tcum - tv, 0))
    l16 = lax.shift_right_arithmetic(jnp.max(ecum), 4)

    def wl(c, _):
        src = pl.ds(c * 16, 16)
        dst = pl.ds(pl.multiple_of(my_wbase + c * 16, 16), 16)
        pltpu.sync_copy(la_v.at[src], sh_lv.at[dst])
        pltpu.sync_copy(la_i.at[src], sh_li.at[dst])
        return 0
    lax.fori_loop(0, my_chunks, wl, 0)
    plsc.subcore_barrier()
    pltpu.sync_copy(sh_lv, lv)
    pltpu.sync_copy(sh_li, li)

    def zo(j, _):
        s = pl.ds(j * 16, 16)
        obi[s] = jnp.zeros((16,), i32)
        obc[s] = jnp.zeros((16,), i32)
        obv[s] = jnp.zeros((16,), jnp.float32)
        return 0
    lax.fori_loop(0, _LCAP // 16, zo, 0)

    def tb(j, run):
        v = key_sel(j)
        m = v == vstar
        mi = m.astype(i32)
        excl = plsc.cumsum(mi) - mi
        trank = my_tiebase + run + excl
        sel = jnp.logical_and(m, trank < big_r)
        pos = jnp.where(sel, obase + a_cnt + trank, 0)
        gi = base + j * 16 + iota
        plsc.store_scatter(obi, [pos], gi, mask=sel)
        plsc.store_scatter(obc, [pos], gca[pl.ds(j * 16, 16)], mask=sel)
        plsc.store_scatter(obv, [pos], mva[pl.ds(j * 16, 16)], mask=sel)
        return run + jnp.max(plsc.all_reduce_population_count(m))
    lax.fori_loop(0, _EV, tb, jnp.int32(0))

    lane0 = iota == 0

    def ab(i, _):
        ivec = jnp.zeros((16,), i32) + i
        cv = _u32(plsc.load_gather(la_v, [ivec]))
        ci = plsc.load_gather(la_i, [ivec])

        def pw(j, acc):
            wv = _u32(lv[pl.ds(j * 16, 16)])
            wi = li[pl.ds(j * 16, 16)]
            hit = jnp.logical_or(
                wv > cv, jnp.logical_and(wv == cv, wi < ci))
            return acc + hit.astype(i32)
        acc = lax.fori_loop(0, l16, pw, jnp.zeros((16,), i32))
        pos = jnp.zeros((16,), i32) + obase + jnp.sum(acc)
        plsc.store_scatter(obi, [pos], ci, mask=lane0)
        plsc.store_scatter(obc, [pos], plsc.load_gather(la_gc, [ivec]), mask=lane0)
        plsc.store_scatter(obv, [pos], plsc.load_gather(la_mv, [ivec]), mask=lane0)
        return 0
    lax.fori_loop(0, n_above, ab, 0)

    pltpu.sync_copy(obi, sh_oi.at[wid])
    pltpu.sync_copy(obc, sh_oc.at[wid])
    pltpu.sync_copy(obv, sh_ov.at[wid])
    plsc.subcore_barrier()

    def merge(sh, mg, stage, out_hbm):
        def msum(off, ln):
            hs = []
            for t in range(_NT):
                hs.append(pltpu.async_copy(sh.at[t], mg.at[pl.ds(t * _LCAP, _LCAP)], msem))
            for h in hs:
                h.wait()

            def sv(v, _):
                a = mg[pl.ds(off + v * 16, 16)]
                for t in range(1, _NT):
                    a = a + mg[pl.ds(t * _LCAP + off + v * 16, 16)]
                stage[pl.ds(v * 16, 16)] = a
                return 0
            lax.fori_loop(0, ln // 16, sv, 0)
            pltpu.sync_copy(stage.at[pl.ds(0, ln)], out_hbm.at[pl.ds(off, ln)])

        @pl.when(fgc)
        def _():
            msum(0, _K_FG)

        @pl.when(jnp.logical_not(fgc))
        def _():
            msum(_K_FG, _K_BG)

    @pl.when(wid == 0)
    def _():
        merge(sh_oi, mgi, lv, oi_hbm)

    @pl.when(wid == 1)
    def _():
        merge(sh_oc, mgi, li, oc_hbm)

    @pl.when(wid == 2)
    def _():
        merge(sh_ov, mgf, stf, ov_hbm)


@jax.jit
def kernel(proposal_boxes, proposal_scores, gt_boxes, gt_classes):
    boxes = jnp.concatenate([proposal_boxes, gt_boxes], axis=0)
    boxes = jnp.pad(boxes, ((0, _PAD - _TOT), (0, 0)))
    scores = jnp.concatenate(
        [proposal_scores, jnp.ones((_M,), jnp.float32)], axis=0)
    scores = jnp.pad(scores, (0, _PAD - _TOT)).reshape(_ROWS, 128)
    cols = [boxes[:, i].reshape(_ROWS, 128) for i in range(4)]
    gt_classes = gt_classes.astype(jnp.int32)

    smem = pl.BlockSpec(memory_space=pltpu.SMEM)
    vf, vb, mv, gc = pl.pallas_call(
        _match_body,
        in_specs=[smem, smem] + [pl.BlockSpec((_ROWS, 128), lambda: (0, 0))] * 5,
        out_specs=[pl.BlockSpec((_ROWS, 128), lambda: (0, 0))] * 4,
        out_shape=(
            jax.ShapeDtypeStruct((_ROWS, 128), jnp.int32),
            jax.ShapeDtypeStruct((_ROWS, 128), jnp.int32),
            jax.ShapeDtypeStruct((_ROWS, 128), jnp.float32),
            jax.ShapeDtypeStruct((_ROWS, 128), jnp.int32),
        ),
    )(gt_boxes, gt_classes, *cols, scores)

    i32 = jnp.int32
    f32 = jnp.float32
    mesh = plsc.VectorSubcoreMesh(core_axis_name="c", subcore_axis_name="s")
    sel = pl.kernel(
        _select_body,
        out_type=(
            jax.ShapeDtypeStruct((_K,), i32),
            jax.ShapeDtypeStruct((_K,), i32),
            jax.ShapeDtypeStruct((_K,), f32),
        ),
        mesh=mesh,
        compiler_params=pltpu.CompilerParams(needs_layout_passes=False),
        scratch_types=[
            pltpu.VMEM((_E,), i32), pltpu.VMEM((_E,), i32),
            pltpu.VMEM((_E,), f32), pltpu.VMEM((_E,), i32),
            pltpu.VMEM((256,), i32), pltpu.VMEM((_NT, 256), i32),
            pltpu.VMEM((352,), i32), pltpu.VMEM((352,), i32),
            pltpu.VMEM((352,), f32), pltpu.VMEM((352,), i32),
            pltpu.VMEM((_LCAP,), i32), pltpu.VMEM((_LCAP,), i32),
            pltpu.VMEM((_LCAP,), f32), pltpu.VMEM((_LCAP,), i32),
            pltpu.VMEM((_NT * _LCAP,), i32), pltpu.VMEM((_NT * _LCAP,), f32),
            pltpu.VMEM((384,), f32), pltpu.SemaphoreType.DMA,
            pltpu.VMEM((16,), i32), pltpu.VMEM((_NT, 16), i32),
            pltpu.VMEM((_LCAP,), i32), pltpu.VMEM((_LCAP,), i32),
            pltpu.VMEM((_LCAP,), f32),
            pltpu.VMEM_SHARED((4, _NT, 256), i32),
            pltpu.VMEM_SHARED((_NT, 16), i32),
            pltpu.VMEM_SHARED((_LCAP,), i32), pltpu.VMEM_SHARED((_LCAP,), i32),
            pltpu.VMEM_SHARED((_NT, _LCAP), i32),
            pltpu.VMEM_SHARED((_NT, _LCAP), i32),
            pltpu.VMEM_SHARED((_NT, _LCAP), f32),
        ],
    )(vf.reshape(_PAD), vb.reshape(_PAD), mv.reshape(_PAD), gc.reshape(_PAD))
    return sel
